# Initial kernel scaffold; baseline (speedup 1.0000x reference)
import jax
import jax.numpy as jnp
from jax.experimental import pallas as pl


def kernel(x, weight):
    raise NotImplementedError("write your pallas kernel here")


# speedup vs baseline: 3.9152x; 3.9152x over previous
import jax
import jax.numpy as jnp
from jax.experimental import pallas as pl

MAXN = 1.0
BLK = 2048


def _body(x_ref, w_ref, o_ref):
    w = w_ref[:]
    sumsq = jnp.sum(w * w, axis=1, keepdims=True)
    norm = jnp.sqrt(sumsq)
    scale = jnp.where(norm > MAXN, MAXN / (norm + 1e-7), 1.0)
    wn = w * scale

    d = wn[1:2] - wn[0:1]
    z = jnp.zeros((1, 64), jnp.float32)
    b = jnp.concatenate(
        [jnp.concatenate([d, z], axis=1), jnp.concatenate([z, d], axis=1)],
        axis=0,
    )
    w0c = jnp.concatenate([wn[0:1], wn[0:1]], axis=1)

    f = x_ref[:].astype(jnp.float32)
    g = jnp.dot(f, b, preferred_element_type=jnp.float32,
                precision=jax.lax.Precision.HIGHEST)
    o_ref[:] = g + w0c


def kernel(x, weight):
    b, l = x.shape
    n2 = (b * l) // 2
    x2 = x.reshape(n2, 2).astype(jnp.int32)
    grid = n2 // BLK
    out2 = pl.pallas_call(
        _body,
        grid=(grid,),
        in_specs=[
            pl.BlockSpec((BLK, 2), lambda i: (i, 0)),
            pl.BlockSpec((2, 64), lambda i: (0, 0)),
        ],
        out_specs=pl.BlockSpec((BLK, 128), lambda i: (i, 0)),
        out_shape=jax.ShapeDtypeStruct((n2, 128), jnp.float32),
    )(x2, weight)
    return out2.reshape(b, l, 64)


# speedup vs baseline: 4.4132x; 1.1272x over previous
import jax
import jax.numpy as jnp
from jax.experimental import pallas as pl

MAXN = 1.0
BLK = 8192


def _body(x_ref, w_ref, o_ref):
    w = w_ref[:]
    sumsq = jnp.sum(w * w, axis=1, keepdims=True)
    norm = jnp.sqrt(sumsq)
    scale = jnp.where(norm > MAXN, MAXN / (norm + 1e-7), 1.0)
    wn = w * scale

    d = wn[1:2] - wn[0:1]
    z = jnp.zeros((1, 64), jnp.float32)
    b = jnp.concatenate(
        [jnp.concatenate([d, z], axis=1), jnp.concatenate([z, d], axis=1)],
        axis=0,
    )
    w0c = jnp.concatenate([wn[0:1], wn[0:1]], axis=1)

    f = x_ref[:].astype(jnp.float32)
    g = jnp.dot(f, b, preferred_element_type=jnp.float32,
                precision=jax.lax.Precision.HIGHEST)
    o_ref[:] = g + w0c


def kernel(x, weight):
    b, l = x.shape
    n2 = (b * l) // 2
    x2 = x.reshape(n2, 2).astype(jnp.int32)
    grid = n2 // BLK
    out2 = pl.pallas_call(
        _body,
        grid=(grid,),
        in_specs=[
            pl.BlockSpec((BLK, 2), lambda i: (i, 0)),
            pl.BlockSpec((2, 64), lambda i: (0, 0)),
        ],
        out_specs=pl.BlockSpec((BLK, 128), lambda i: (i, 0)),
        out_shape=jax.ShapeDtypeStruct((n2, 128), jnp.float32),
    )(x2, weight)
    return out2.reshape(b, l, 64)


# speedup vs baseline: 5.5629x; 1.2605x over previous
import jax
import jax.numpy as jnp
from jax import lax
from jax.experimental import pallas as pl
from jax.experimental.pallas import tpu as pltpu
from jax.experimental.pallas import tpu_sc as plsc

MAXN = 1.0
NCORES = 2
NSUB = 16
NW = NCORES * NSUB
CHUNK = 512
PAIRS = CHUNK // 2


def _rsqrt_nr(s):
    i = lax.bitcast_convert_type(s, jnp.int32)
    y = lax.bitcast_convert_type(jnp.int32(0x5F3759DF) - (i >> 1), jnp.float32)
    for _ in range(3):
        y = y * (1.5 - 0.5 * s * y * y)
    return y


def _lane_sum(v):
    idx = lax.iota(jnp.int32, 16)
    dnums = lax.GatherDimensionNumbers(
        offset_dims=(), collapsed_slice_dims=(0,), start_index_map=(0,)
    )
    for sh in (8, 4, 2, 1):
        perm = lax.gather(
            v, (idx ^ sh)[:, None], dnums, (1,),
            mode=lax.GatherScatterMode.PROMISE_IN_BOUNDS,
        )
        v = v + perm
    return v


def _sc_body(idx_hbm, w_hbm, out_hbm, w_v, wn_v, idx_v, out_v, sem_o):
    wid = lax.axis_index("s") * NCORES + lax.axis_index("c")
    rows = idx_hbm.shape[0] // NW
    nchunks = rows // CHUNK
    base = wid * rows

    pltpu.sync_copy(w_hbm, w_v)
    for t in range(2):
        qs = [w_v[t, pl.ds(16 * q, 16)] for q in range(4)]
        ss = qs[0] * qs[0] + qs[1] * qs[1] + qs[2] * qs[2] + qs[3] * qs[3]
        sv = _lane_sum(ss)
        normv = sv * _rsqrt_nr(sv)
        scalev = jnp.where(normv > MAXN, MAXN / (normv + 1e-7), 1.0)
        for q in range(4):
            wn_v[t, pl.ds(16 * q, 16)] = qs[q] * scalev

    pairs = PAIRS
    pbase = wid * (rows // 2)

    @pl.loop(0, nchunks, step=2)
    def _chunks(g):
        for b in range(2):
            gg = g + b
            rowbase = base + gg * CHUNK
            prowbase = pbase + gg * pairs

            @pl.when(gg >= 2)
            def _():
                pltpu.make_async_copy(
                    out_v.at[b], out_hbm.at[pl.ds(pbase, pairs)], sem_o.at[b]
                ).wait()

            pltpu.sync_copy(idx_hbm.at[pl.ds(rowbase, CHUNK)], idx_v.at[b])
            w0 = [wn_v[0, pl.ds(16 * q, 16)] for q in range(4)]
            w1 = [wn_v[1, pl.ds(16 * q, 16)] for q in range(4)]

            @pl.loop(0, CHUNK // 16)
            def _rows(r16):
                rb = r16 * 16
                iv = idx_v[b, pl.ds(rb, 16)]
                for lane in range(16):
                    pred = iv[lane] > 0
                    pr = rb // 2 + lane // 2
                    half = 64 * (lane % 2)
                    for q in range(4):
                        out_v[b, pr, pl.ds(half + 16 * q, 16)] = jnp.where(
                            pred, w1[q], w0[q]
                        )

            pltpu.async_copy(
                out_v.at[b], out_hbm.at[pl.ds(prowbase, pairs)], sem_o.at[b]
            )

    for b in range(2):
        pltpu.make_async_copy(
            out_v.at[b], out_hbm.at[pl.ds(pbase, pairs)], sem_o.at[b]
        ).wait()


def kernel(x, weight):
    bsz, l = x.shape
    n = bsz * l
    xf = x.reshape(n).astype(jnp.int32)
    mesh = plsc.VectorSubcoreMesh(core_axis_name="c", subcore_axis_name="s")
    sc = pl.kernel(
        _sc_body,
        out_type=jax.ShapeDtypeStruct((n // 2, 128), jnp.float32),
        mesh=mesh,
        scratch_types=[
            pltpu.VMEM((2, 64), jnp.float32),
            pltpu.VMEM((2, 64), jnp.float32),
            pltpu.VMEM((2, CHUNK), jnp.int32),
            pltpu.VMEM((2, PAIRS, 128), jnp.float32),
            pltpu.SemaphoreType.DMA((2,)),
        ],
    )
    out = sc(xf, weight)
    return out.reshape(bsz, l, 64)


# speedup vs baseline: 5.6628x; 1.0180x over previous
import jax
import jax.numpy as jnp
from jax import lax
from jax.experimental import pallas as pl
from jax.experimental.pallas import tpu as pltpu
from jax.experimental.pallas import tpu_sc as plsc

MAXN = 1.0
NCORES = 2
NSUB = 16
NW = NCORES * NSUB
CX = 8
HX = CX // 2
GROUPS = [(s, 0) for s in range(0, 192, 16)] + [(184, 8)]


def _rsqrt_nr(s):
    i = lax.bitcast_convert_type(s, jnp.int32)
    y = lax.bitcast_convert_type(jnp.int32(0x5F3759DF) - (i >> 1), jnp.float32)
    for _ in range(3):
        y = y * (1.5 - 0.5 * s * y * y)
    return y


def _lane_sum(v):
    idx = lax.iota(jnp.int32, 16)
    dnums = lax.GatherDimensionNumbers(
        offset_dims=(), collapsed_slice_dims=(0,), start_index_map=(0,)
    )
    for sh in (8, 4, 2, 1):
        perm = lax.gather(
            v, (idx ^ sh)[:, None], dnums, (1,),
            mode=lax.GatherScatterMode.PROMISE_IN_BOUNDS,
        )
        v = v + perm
    return v


def _sc_body(x_hbm, w_hbm, out_hbm, w_v, wn_v, idx_v, out_v, sem_o):
    wid = lax.axis_index("s") * NCORES + lax.axis_index("c")
    ncols = x_hbm.shape[1]
    xrows = x_hbm.shape[0] // NW
    nchunks = xrows // CX
    xbase = wid * xrows
    ppr = (ncols * 64) // 128
    hpairs = HX * ppr

    pltpu.sync_copy(w_hbm, w_v)
    for t in range(2):
        qs = [w_v[t, pl.ds(16 * q, 16)] for q in range(4)]
        ss = qs[0] * qs[0] + qs[1] * qs[1] + qs[2] * qs[2] + qs[3] * qs[3]
        sv = _lane_sum(ss)
        normv = sv * _rsqrt_nr(sv)
        scalev = jnp.where(normv > MAXN, MAXN / (normv + 1e-7), 1.0)
        for q in range(4):
            wn_v[t, pl.ds(16 * q, 16)] = qs[q] * scalev

    @pl.loop(0, nchunks)
    def _chunks(g):
        xrow0 = xbase + g * CX
        pltpu.sync_copy(x_hbm.at[pl.ds(xrow0, CX)], idx_v)
        w0 = [wn_v[0, pl.ds(16 * q, 16)] for q in range(4)]
        w1 = [wn_v[1, pl.ds(16 * q, 16)] for q in range(4)]

        for h in range(2):
            @pl.when(g >= 1)
            def _():
                pltpu.make_async_copy(
                    out_v.at[h], out_hbm.at[pl.ds(0, hpairs)], sem_o.at[h]
                ).wait()

            @pl.loop(0, HX)
            def _xr(xr):
                fbase = xr * ncols
                for st, lane_lo in GROUPS:
                    iv = idx_v[h * HX + xr, pl.ds(st, 16)]
                    for lane in range(lane_lo, 16):
                        f = fbase + st + lane
                        pred = iv[lane] > 0
                        half = 64 * ((st + lane) % 2)
                        for q in range(4):
                            out_v[h, f // 2, pl.ds(half + 16 * q, 16)] = jnp.where(
                                pred, w1[q], w0[q]
                            )

            prow0 = (xrow0 + h * HX) * ppr
            pltpu.async_copy(
                out_v.at[h], out_hbm.at[pl.ds(prow0, hpairs)], sem_o.at[h]
            )

    for h in range(2):
        pltpu.make_async_copy(
            out_v.at[h], out_hbm.at[pl.ds(0, hpairs)], sem_o.at[h]
        ).wait()


def kernel(x, weight):
    bsz, l = x.shape
    n = bsz * l
    ppr = (l * 64) // 128
    mesh = plsc.VectorSubcoreMesh(core_axis_name="c", subcore_axis_name="s")
    sc = pl.kernel(
        _sc_body,
        out_type=jax.ShapeDtypeStruct((n // 2, 128), jnp.float32),
        mesh=mesh,
        scratch_types=[
            pltpu.VMEM((2, 64), jnp.float32),
            pltpu.VMEM((2, 64), jnp.float32),
            pltpu.VMEM((CX, 200), jnp.int32),
            pltpu.VMEM((2, HX * ppr, 128), jnp.float32),
            pltpu.SemaphoreType.DMA((2,)),
        ],
    )
    out = sc(x.astype(jnp.int32), weight)
    return out.reshape(bsz, l, 64)


# speedup vs baseline: 7.1595x; 1.2643x over previous
import jax
import jax.numpy as jnp
from jax import lax
from jax.experimental import pallas as pl
from jax.experimental.pallas import tpu as pltpu
from jax.experimental.pallas import tpu_sc as plsc

MAXN = 1.0
NCORES = 2
NSUB = 16
NW = NCORES * NSUB
JB = 40


def _rsqrt_nr(s):
    i = lax.bitcast_convert_type(s, jnp.int32)
    y = lax.bitcast_convert_type(jnp.int32(0x5F3759DF) - (i >> 1), jnp.float32)
    for _ in range(3):
        y = y * (1.5 - 0.5 * s * y * y)
    return y


def _lane_sum(v):
    idx = lax.iota(jnp.int32, 16)
    dnums = lax.GatherDimensionNumbers(
        offset_dims=(), collapsed_slice_dims=(0,), start_index_map=(0,)
    )
    for sh in (8, 4, 2, 1):
        perm = lax.gather(
            v, (idx ^ sh)[:, None], dnums, (1,),
            mode=lax.GatherScatterMode.PROMISE_IN_BOUNDS,
        )
        v = v + perm
    return v


def _sc_body(xt_hbm, w_hbm, y_hbm, w_v, spl0, spl1, xbuf, out_v, sem_o):
    wid = lax.axis_index("s") * NCORES + lax.axis_index("c")
    ncols = xt_hbm.shape[0]
    nb = xt_hbm.shape[1] // NW
    i0 = wid * nb
    nchunks = ncols // JB

    pltpu.sync_copy(w_hbm, w_v)
    wn = []
    for t in range(2):
        qs = [w_v[t, pl.ds(16 * q, 16)] for q in range(4)]
        ss = qs[0] * qs[0] + qs[1] * qs[1] + qs[2] * qs[2] + qs[3] * qs[3]
        sv = _lane_sum(ss)
        normv = sv * _rsqrt_nr(sv)
        scalev = jnp.where(normv > MAXN, MAXN / (normv + 1e-7), 1.0)
        wn.append([qs[q] * scalev for q in range(4)])
    for k in range(64):
        q, ln = k // 16, k % 16
        spl0[k // 8, pl.ds(16 * (k % 8), 16)] = jnp.full((16,), wn[0][q][ln])
        spl1[k // 8, pl.ds(16 * (k % 8), 16)] = jnp.full((16,), wn[1][q][ln])

    @pl.loop(0, nchunks)
    def _c(c):
        jg0 = c * JB
        pltpu.sync_copy(
            xt_hbm.at[pl.ds(jg0, JB), pl.ds(i0, nb)], xbuf
        )

        @pl.loop(0, JB, step=2)
        def _j(jl):
            for b in range(2):
                j = jg0 + jl + b

                @pl.when(j >= 2)
                def _():
                    pltpu.make_async_copy(
                        out_v.at[b],
                        y_hbm.at[0, :, pl.ds(0, nb)],
                        sem_o.at[b],
                    ).wait()

                @pl.loop(0, 64)
                def _k(k):
                    s0 = spl0[k // 8, pl.ds(16 * (k % 8), 16)]
                    s1 = spl1[k // 8, pl.ds(16 * (k % 8), 16)]

                    @pl.loop(0, nb // 16, unroll=4)
                    def _ib(ib):
                        xc = xbuf[jl + b, pl.ds(16 * ib, 16)]
                        out_v[b, k, pl.ds(16 * ib, 16)] = jnp.where(
                            xc > 0, s1, s0
                        )

                pltpu.async_copy(
                    out_v.at[b], y_hbm.at[j, :, pl.ds(i0, nb)], sem_o.at[b]
                )

    for b in range(2):
        pltpu.make_async_copy(
            out_v.at[b], y_hbm.at[0, :, pl.ds(0, nb)], sem_o.at[b]
        ).wait()


def kernel(x, weight):
    bsz, l = x.shape
    nb = bsz // NW
    mesh = plsc.VectorSubcoreMesh(core_axis_name="c", subcore_axis_name="s")
    sc = pl.kernel(
        _sc_body,
        out_type=jax.ShapeDtypeStruct((l, 64, bsz), jnp.float32),
        mesh=mesh,
        compiler_params=pltpu.CompilerParams(use_tc_tiling_on_sc=True),
        scratch_types=[
            pltpu.VMEM((2, 64), jnp.float32),
            pltpu.VMEM((8, 128), jnp.float32),
            pltpu.VMEM((8, 128), jnp.float32),
            pltpu.VMEM((JB, nb), jnp.int32),
            pltpu.VMEM((2, 64, nb), jnp.float32),
            pltpu.SemaphoreType.DMA((2,)),
        ],
    )
    y = sc(x.T.astype(jnp.int32), weight)
    return y.transpose(2, 0, 1)
